# baseline (speedup 1.0000x reference)
import functools

import jax
import jax.numpy as jnp
from jax import lax
from jax.experimental import pallas as pl
from jax.experimental.pallas import tpu as pltpu
from jax.experimental.pallas import tpu_sc as plsc

_D = 64
_DP = 128
_SCALE = 10.0
_NW = 32
_G = 4
_NBUF = 8


def _build(B, S):
    b_per_w = B // _NW
    i_per_w = b_per_w * S
    gsz = _G * S
    n_groups = b_per_w // _G
    n_rounds = n_groups // _NBUF
    assert B % _NW == 0 and b_per_w % (_G * _NBUF) == 0 and gsz <= 128
    mesh = plsc.VectorSubcoreMesh(core_axis_name="c", subcore_axis_name="s")

    @functools.partial(
        pl.kernel,
        mesh=mesh,
        compiler_params=pltpu.CompilerParams(use_tc_tiling_on_sc=False),
        out_type=jax.ShapeDtypeStruct((B, 32, _DP), jnp.float32),
        scratch_types=[
            pltpu.VMEM((i_per_w,), jnp.int32),
            pltpu.VMEM((_NBUF, gsz, _D), jnp.float32),
            pltpu.SemaphoreType.DMA,
        ]
        + [pltpu.SemaphoreType.DMA] * _NBUF
        + [pltpu.SemaphoreType.DMA] * _NBUF,
    )
    def embed(table_hbm, idx_hbm, out_hbm, idx_v, rows_v, isem, *bsems):
        gsem = bsems[:_NBUF]
        ssem = bsems[_NBUF:]
        wid = lax.axis_index("s") * 2 + lax.axis_index("c")
        base = wid * b_per_w
        pltpu.async_copy(idx_hbm.at[pl.ds(base * S, i_per_w)], idx_v, isem).wait()

        def start_gather(g, b):
            pltpu.async_copy(
                table_hbm.at[idx_v.at[pl.ds(g * gsz, gsz)]], rows_v.at[b], gsem[b]
            )

        def wait_gather(b):
            pltpu.make_async_copy(
                table_hbm.at[idx_v.at[pl.ds(0, gsz)]], rows_v.at[b], gsem[b]
            ).wait()

        def start_stores(g, b):
            for i in range(_G):
                pltpu.async_copy(
                    rows_v.at[b, pl.ds(i * S, S)],
                    out_hbm.at[base + g * _G + i, pl.ds(0, S), pl.ds(0, _D)],
                    ssem[b],
                )

        def wait_stores(b):
            for i in range(_G):
                pltpu.make_async_copy(
                    out_hbm.at[0, pl.ds(0, S), pl.ds(0, _D)],
                    rows_v.at[b, pl.ds(i * S, S)],
                    ssem[b],
                ).wait()

        def scale(b):
            def cbody(j0, _, b=b):
                j = j0 * 2
                for dj in range(2):
                    for c in range(_D // 16):
                        sl = pl.ds(c * 16, 16)
                        rows_v[b, j + dj, sl] = rows_v[b, j + dj, sl] * _SCALE
                return _

            lax.fori_loop(0, gsz // 2, cbody, None)

        for b in range(_NBUF):
            start_gather(b, b)

        def round_body(r, _):
            g0 = r * _NBUF
            for b in range(_NBUF):
                wait_gather(b)
                scale(b)
                start_stores(g0 + b, b)

            @pl.when(r < n_rounds - 1)
            def _refill():
                for b in range(_NBUF):
                    wait_stores(b)
                    start_gather(g0 + _NBUF + b, b)

            @pl.when(r == n_rounds - 1)
            def _drain():
                for b in range(_NBUF):
                    wait_stores(b)

            return _

        lax.fori_loop(0, n_rounds, round_body, None)

    return embed


def kernel(x, weight):
    B, S = x.shape
    out2 = _build(B, S)(weight, x.astype(jnp.int32).reshape(-1))
    return out2[:, :S, :_D]


# speedup vs baseline: 1.0033x; 1.0033x over previous
import functools

import jax
import jax.numpy as jnp
from jax import lax
from jax.experimental import pallas as pl
from jax.experimental.pallas import tpu as pltpu
from jax.experimental.pallas import tpu_sc as plsc

_D = 64
_DP = 128
_SCALE = 10.0
_NW = 32
_G = 4
_NBUF = 8


def _build(B, S):
    b_per_w = B // _NW
    i_per_w = b_per_w * S
    gsz = _G * S
    n_groups = b_per_w // _G
    n_rounds = n_groups // _NBUF
    assert B % _NW == 0 and b_per_w % (_G * _NBUF) == 0 and gsz <= 128
    mesh = plsc.VectorSubcoreMesh(core_axis_name="c", subcore_axis_name="s")

    @functools.partial(
        pl.kernel,
        mesh=mesh,
        compiler_params=pltpu.CompilerParams(use_tc_tiling_on_sc=False),
        out_type=jax.ShapeDtypeStruct((B, (S + 7) // 8 * 8, _DP), jnp.float32),
        scratch_types=[
            pltpu.VMEM((i_per_w,), jnp.int32),
            pltpu.VMEM((_NBUF, gsz, _D), jnp.float32),
            pltpu.SemaphoreType.DMA,
        ]
        + [pltpu.SemaphoreType.DMA] * _NBUF
        + [pltpu.SemaphoreType.DMA] * _NBUF,
    )
    def embed(table_hbm, idx_hbm, out_hbm, idx_v, rows_v, isem, *bsems):
        gsem = bsems[:_NBUF]
        ssem = bsems[_NBUF:]
        wid = lax.axis_index("s") * 2 + lax.axis_index("c")
        base = wid * b_per_w
        pltpu.async_copy(idx_hbm.at[pl.ds(base * S, i_per_w)], idx_v, isem).wait()

        def start_gather(g, b):
            pltpu.async_copy(
                table_hbm.at[idx_v.at[pl.ds(g * gsz, gsz)]], rows_v.at[b], gsem[b]
            )

        def wait_gather(b):
            pltpu.make_async_copy(
                table_hbm.at[idx_v.at[pl.ds(0, gsz)]], rows_v.at[b], gsem[b]
            ).wait()

        def start_stores(g, b):
            for i in range(_G):
                pltpu.async_copy(
                    rows_v.at[b, pl.ds(i * S, S)],
                    out_hbm.at[base + g * _G + i, pl.ds(0, S), pl.ds(0, _D)],
                    ssem[b],
                )

        def wait_stores(b):
            for i in range(_G):
                pltpu.make_async_copy(
                    out_hbm.at[0, pl.ds(0, S), pl.ds(0, _D)],
                    rows_v.at[b, pl.ds(i * S, S)],
                    ssem[b],
                ).wait()

        def scale(b):
            def cbody(j0, _, b=b):
                j = j0 * 2
                for dj in range(2):
                    for c in range(_D // 16):
                        sl = pl.ds(c * 16, 16)
                        rows_v[b, j + dj, sl] = rows_v[b, j + dj, sl] * _SCALE
                return _

            lax.fori_loop(0, gsz // 2, cbody, None)

        for b in range(_NBUF):
            start_gather(b, b)

        def round_body(r, _):
            g0 = r * _NBUF
            for b in range(_NBUF):
                wait_gather(b)
                scale(b)
                start_stores(g0 + b, b)

            @pl.when(r < n_rounds - 1)
            def _refill():
                for b in range(_NBUF):
                    wait_stores(b)
                    start_gather(g0 + _NBUF + b, b)

            @pl.when(r == n_rounds - 1)
            def _drain():
                for b in range(_NBUF):
                    wait_stores(b)

            return _

        lax.fori_loop(0, n_rounds, round_body, None)

    return embed


def kernel(x, weight):
    B, S = x.shape
    out2 = _build(B, S)(weight, x.astype(jnp.int32).reshape(-1))
    return out2[:, :S, :_D]
